# baseline (speedup 1.0000x reference)
import functools

import jax
import jax.numpy as jnp
from jax import lax
from jax.experimental import pallas as pl
from jax.experimental.pallas import tpu as pltpu
from jax.experimental.pallas import tpu_sc as plsc

B, C, N = 16, 3, 4096
LVL = 8
P = 64
LANES = 16
NCHUNK = N // LANES
NSROW = B * C
NREC = NSROW + P


def _iota():
    return lax.broadcasted_iota(jnp.int32, (LANES,), 0)


def _popcount32(v):
    v = v - (lax.shift_right_logical(v, 1) & 0x55555555)
    v = (v & 0x33333333) + (lax.shift_right_logical(v, 2) & 0x33333333)
    v = (v + lax.shift_right_logical(v, 4)) & 0x0F0F0F0F
    return lax.shift_right_logical(v * jnp.int32(0x01010101), 24)


def _set_bit(lo, hi, j):
    lo = lo | jnp.where(j < 32, lax.shift_left(jnp.int32(1), jnp.minimum(j, 31)), 0)
    hi = hi | jnp.where(j >= 32, lax.shift_left(jnp.int32(1), jnp.maximum(j - 32, 0)), 0)
    return lo, hi


def _tc_quant(x48, lev48):
    def body(x_ref, lev_ref, q_ref):
        x = x_ref[...]
        best = jnp.abs(x - lev_ref[:, 0:1])
        q = jnp.zeros(x.shape, jnp.int32)
        for l in range(1, LVL):
            d = jnp.abs(x - lev_ref[:, l : l + 1])
            take = d < best
            q = jnp.where(take, jnp.int32(l), q)
            best = jnp.where(take, d, best)
        q_ref[...] = q

    return pl.pallas_call(
        body,
        out_shape=jax.ShapeDtypeStruct((NSROW, N), jnp.int32),
    )(x48, lev48)


def _make_sc(interpret=False):
    mesh = plsc.VectorSubcoreMesh(core_axis_name="c", subcore_axis_name="s",
                                  num_cores=1)

    @functools.partial(
        pl.kernel,
        out_type=jax.ShapeDtypeStruct((B, P), jnp.float32),
        mesh=mesh,
        scratch_types=[
            pltpu.VMEM((N + 128,), jnp.int32),
            pltpu.VMEM((N,), jnp.int32),
            pltpu.VMEM((N + 128,), jnp.int32),
            pltpu.VMEM((128,), jnp.int32),
            pltpu.VMEM((LANES,), jnp.int32),
            pltpu.VMEM((LANES,), jnp.float32),
            pltpu.VMEM_SHARED((NREC * LANES,), jnp.int32),
            pltpu.VMEM((NREC * LANES,), jnp.int32),
        ],
        compiler_params=pltpu.CompilerParams(needs_layout_passes=False),
        interpret=interpret,
    )
    def ccp(q_hbm, curve_hbm, pmap_hbm, out_hbm,
            curve_v, qrow_v, prow_v, pres_v,
            stage_i, stage_f, recs_sh, recs_v):
        sid = lax.axis_index("s")
        iota = _iota()
        ones16 = jnp.full((LANES,), 1, jnp.int32)
        zeros16 = jnp.zeros((LANES,), jnp.int32)

        pltpu.sync_copy(curve_hbm, curve_v.at[pl.ds(0, N)])
        curve_v[pl.ds(N, LANES)] = zeros16

        def clear_presence():
            for j in range(P // LANES):
                pres_v[pl.ds(j * LANES, LANES)] = zeros16

        def emit_record(row, first, last):
            parts = []
            for j in range(P // LANES):
                pj = pres_v[pl.ds(j * LANES, LANES)]
                parts.append(jnp.sum(lax.shift_left(pj, iota)))
            lo = parts[0] | lax.shift_left(parts[1], 16)
            hi = parts[2] | lax.shift_left(parts[3], 16)
            rec = (jnp.where(iota == 0, lo, 0)
                   | jnp.where(iota == 1, hi, 0)
                   | jnp.where(iota == 2, first, 0)
                   | jnp.where(iota == 3, last, 0))
            stage_i[...] = rec
            pltpu.sync_copy(stage_i, recs_sh.at[pl.ds(row * LANES, LANES)])

        def do_s_row(r):
            pltpu.sync_copy(q_hbm.at[r], qrow_v)
            clear_presence()

            @plsc.parallel_loop(0, NCHUNK, unroll=4)
            def _codes(i):
                base = i * LANES
                i0 = curve_v[pl.ds(base, LANES)]
                i1 = curve_v[pl.ds(base + 1, LANES)]
                q0 = plsc.load_gather(qrow_v, [i0])
                q1 = plsc.load_gather(qrow_v, [i1])
                code = q0 * LVL + q1
                msk = (base + iota) < (N - 1)
                plsc.store_scatter(pres_v, [code], ones16, mask=msk)

            q_head = plsc.load_gather(qrow_v, [curve_v[pl.ds(0, LANES)]])
            q_tail = plsc.load_gather(qrow_v, [curve_v[pl.ds(N - LANES, LANES)]])
            emit_record(r, q_head[0], q_tail[LANES - 1])

        def do_p_row(p):
            pltpu.sync_copy(pmap_hbm.at[p], prow_v.at[pl.ds(0, N)])
            prow_v[pl.ds(N, LANES)] = zeros16
            clear_presence()

            @plsc.parallel_loop(0, NCHUNK, unroll=4)
            def _codes(i):
                base = i * LANES
                a = prow_v[pl.ds(base, LANES)]
                b2 = prow_v[pl.ds(base + 1, LANES)]
                code = a * LVL + b2
                msk = (base + iota) < (N - 1)
                plsc.store_scatter(pres_v, [code], ones16, mask=msk)

            p_head = prow_v[pl.ds(0, LANES)]
            p_tail = prow_v[pl.ds(N - LANES, LANES)]
            emit_record(NSROW + p, p_head[0], p_tail[LANES - 1])

        def s_loop(k, carry):
            do_s_row(sid + 16 * k)
            return carry

        lax.fori_loop(0, 3, s_loop, None)

        def p_loop(k, carry):
            do_p_row(sid * 4 + k)
            return carry

        lax.fori_loop(0, 4, p_loop, None)

        plsc.subcore_barrier()

        pltpu.sync_copy(recs_sh, recs_v)

        b = sid
        r0 = b * C
        rec0 = recs_v[pl.ds(r0 * LANES, LANES)]
        rec1 = recs_v[pl.ds((r0 + 1) * LANES, LANES)]
        rec2 = recs_v[pl.ds((r0 + 2) * LANES, LANES)]
        lo = rec0[0] | rec1[0] | rec2[0]
        hi = rec0[1] | rec1[1] | rec2[1]
        j1 = rec0[3] * LVL + rec1[2]
        j2 = rec1[3] * LVL + rec2[2]
        lo, hi = _set_bit(lo, hi, j1)
        lo, hi = _set_bit(lo, hi, j2)
        s_last = rec2[3]
        lo_v = jnp.broadcast_to(lo, (LANES,))
        hi_v = jnp.broadcast_to(hi, (LANES,))
        cs = (_popcount32(lo_v) + _popcount32(hi_v)).astype(jnp.float32)

        def combine(pv, carry):
            rows = (NSROW + pv * LANES + iota) * LANES
            p_lo = plsc.load_gather(recs_v, [rows])
            p_hi = plsc.load_gather(recs_v, [rows + 1])
            p_first = plsc.load_gather(recs_v, [rows + 2])

            u_lo = p_lo | lo_v
            u_hi = p_hi | hi_v
            pc = _popcount32(u_lo) + _popcount32(u_hi)
            j = s_last * LVL + p_first
            bit = jnp.where(
                j < 32,
                lax.shift_right_logical(u_lo, jnp.minimum(j, 31)) & 1,
                lax.shift_right_logical(u_hi, jnp.maximum(j - 32, 0)) & 1,
            )
            csp = (pc + 1 - bit).astype(jnp.float32)
            cp = (_popcount32(p_lo) + _popcount32(p_hi)).astype(jnp.float32)
            ncd = (csp - jnp.minimum(cs, cp)) / jnp.maximum(cs, cp)
            stage_f[...] = ncd
            pltpu.sync_copy(stage_f, out_hbm.at[b, pl.ds(pv * LANES, LANES)])
            return carry

        lax.fori_loop(0, 4, combine, None)

    return ccp


@functools.cache
def _get_sc():
    return _make_sc()


def kernel(x, curve, levels, pmap):
    x48 = x.reshape(NSROW, N)
    lev48 = jnp.tile(levels.astype(jnp.float32), (B, 1))
    pf = pmap.reshape(-1, pmap.shape[-1]).astype(jnp.int32)
    q48 = _tc_quant(x48, lev48)
    return _get_sc()(q48, curve.astype(jnp.int32), pf)


# speedup vs baseline: 1.0905x; 1.0905x over previous
import functools

import jax
import jax.numpy as jnp
from jax import lax
from jax.experimental import pallas as pl
from jax.experimental.pallas import tpu as pltpu
from jax.experimental.pallas import tpu_sc as plsc

B, C, N = 16, 3, 4096
LVL = 8
P = 64
LANES = 16
NCHUNK = N // LANES
NSROW = B * C
NREC = NSROW + P


def _iota():
    return lax.broadcasted_iota(jnp.int32, (LANES,), 0)


def _popcount32(v):
    v = v - (lax.shift_right_logical(v, 1) & 0x55555555)
    v = (v & 0x33333333) + (lax.shift_right_logical(v, 2) & 0x33333333)
    v = (v + lax.shift_right_logical(v, 4)) & 0x0F0F0F0F
    return lax.shift_right_logical(v * jnp.int32(0x01010101), 24)


def _set_bit(lo, hi, j):
    lo = lo | jnp.where(j < 32, lax.shift_left(jnp.int32(1), jnp.minimum(j, 31)), 0)
    hi = hi | jnp.where(j >= 32, lax.shift_left(jnp.int32(1), jnp.maximum(j - 32, 0)), 0)
    return lo, hi


def _tc_quant(x48, lev48):
    def body(x_ref, lev_ref, q_ref):
        x = x_ref[...]
        best = jnp.abs(x - lev_ref[:, 0:1])
        q = jnp.zeros(x.shape, jnp.int32)
        for l in range(1, LVL):
            d = jnp.abs(x - lev_ref[:, l : l + 1])
            take = d < best
            q = jnp.where(take, jnp.int32(l), q)
            best = jnp.where(take, d, best)
        q_ref[...] = q

    return pl.pallas_call(
        body,
        out_shape=jax.ShapeDtypeStruct((NSROW, N), jnp.int32),
    )(x48, lev48)


def _make_sc(interpret=False):
    mesh = plsc.VectorSubcoreMesh(core_axis_name="c", subcore_axis_name="s",
                                  num_cores=1)

    @functools.partial(
        pl.kernel,
        out_type=jax.ShapeDtypeStruct((B, P), jnp.float32),
        mesh=mesh,
        scratch_types=[
            pltpu.VMEM((N + 128,), jnp.int32),
            pltpu.VMEM((N,), jnp.int32),
            pltpu.VMEM((N,), jnp.int32),
            pltpu.VMEM((N + 128,), jnp.int32),
            pltpu.VMEM((N + 128,), jnp.int32),
            pltpu.VMEM((128,), jnp.int32),
            pltpu.VMEM((LANES,), jnp.int32),
            pltpu.VMEM((LANES,), jnp.float32),
            pltpu.VMEM_SHARED((NREC * LANES,), jnp.int32),
            pltpu.VMEM((NREC * LANES,), jnp.int32),
            pltpu.SemaphoreType.DMA,
            pltpu.SemaphoreType.DMA,
            pltpu.SemaphoreType.DMA,
            pltpu.SemaphoreType.DMA,
        ],
        compiler_params=pltpu.CompilerParams(needs_layout_passes=False),
        interpret=interpret,
    )
    def ccp(q_hbm, curve_hbm, pmap_hbm, out_hbm,
            curve_v, qrow0, qrow1, prow0, prow1, pres_v,
            stage_i, stage_f, recs_sh, recs_v,
            sem_s0, sem_s1, sem_p0, sem_p1):
        sid = lax.axis_index("s")
        iota = _iota()
        ones16 = jnp.full((LANES,), 1, jnp.int32)
        zeros16 = jnp.zeros((LANES,), jnp.int32)

        pltpu.sync_copy(curve_hbm, curve_v.at[pl.ds(0, N)])
        curve_v[pl.ds(N, LANES)] = zeros16

        def clear_presence():
            for j in range(P // LANES):
                pres_v[pl.ds(j * LANES, LANES)] = zeros16

        def emit_record(row, first, last):
            parts = []
            for j in range(P // LANES):
                pj = pres_v[pl.ds(j * LANES, LANES)]
                parts.append(jnp.sum(lax.shift_left(pj, iota)))
            lo = parts[0] | lax.shift_left(parts[1], 16)
            hi = parts[2] | lax.shift_left(parts[3], 16)
            rec = (jnp.where(iota == 0, lo, 0)
                   | jnp.where(iota == 1, hi, 0)
                   | jnp.where(iota == 2, first, 0)
                   | jnp.where(iota == 3, last, 0))
            stage_i[...] = rec
            pltpu.sync_copy(stage_i, recs_sh.at[pl.ds(row * LANES, LANES)])

        def do_s_row(qrow_v, r):
            clear_presence()

            @plsc.parallel_loop(0, NCHUNK, unroll=4)
            def _codes(i):
                base = i * LANES
                i0 = curve_v[pl.ds(base, LANES)]
                i1 = curve_v[pl.ds(base + 1, LANES)]
                q0 = plsc.load_gather(qrow_v, [i0])
                q1 = plsc.load_gather(qrow_v, [i1])
                code = q0 * LVL + q1
                msk = (base + iota) < (N - 1)
                plsc.store_scatter(pres_v, [code], ones16, mask=msk)

            q_head = plsc.load_gather(qrow_v, [curve_v[pl.ds(0, LANES)]])
            q_tail = plsc.load_gather(qrow_v, [curve_v[pl.ds(N - LANES, LANES)]])
            emit_record(r, q_head[0], q_tail[LANES - 1])

        def do_p_row(prow_v, p):
            clear_presence()

            @plsc.parallel_loop(0, NCHUNK, unroll=4)
            def _codes(i):
                base = i * LANES
                a = prow_v[pl.ds(base, LANES)]
                b2 = prow_v[pl.ds(base + 1, LANES)]
                code = a * LVL + b2
                msk = (base + iota) < (N - 1)
                plsc.store_scatter(pres_v, [code], ones16, mask=msk)

            p_head = prow_v[pl.ds(0, LANES)]
            p_tail = prow_v[pl.ds(N - LANES, LANES)]
            emit_record(NSROW + p, p_head[0], p_tail[LANES - 1])

        prow0[pl.ds(N, LANES)] = zeros16
        prow1[pl.ds(N, LANES)] = zeros16
        s_bufs = (qrow0, qrow1)
        s_sems = (sem_s0, sem_s1)
        dmas = {0: pltpu.async_copy(q_hbm.at[sid], qrow0, sem_s0)}
        for k in range(3):
            if k < 2:
                dmas[k + 1] = pltpu.async_copy(
                    q_hbm.at[sid + 16 * (k + 1)],
                    s_bufs[(k + 1) % 2], s_sems[(k + 1) % 2])
            dmas[k].wait()
            do_s_row(s_bufs[k % 2], sid + 16 * k)

        p_bufs = (prow0, prow1)
        p_sems = (sem_p0, sem_p1)
        pdmas = {0: pltpu.async_copy(
            pmap_hbm.at[sid * 4], prow0.at[pl.ds(0, N)], sem_p0)}
        for k in range(4):
            if k < 3:
                pdmas[k + 1] = pltpu.async_copy(
                    pmap_hbm.at[sid * 4 + k + 1],
                    p_bufs[(k + 1) % 2].at[pl.ds(0, N)], p_sems[(k + 1) % 2])
            pdmas[k].wait()
            do_p_row(p_bufs[k % 2], sid * 4 + k)

        plsc.subcore_barrier()

        pltpu.sync_copy(recs_sh, recs_v)

        b = sid
        r0 = b * C
        rec0 = recs_v[pl.ds(r0 * LANES, LANES)]
        rec1 = recs_v[pl.ds((r0 + 1) * LANES, LANES)]
        rec2 = recs_v[pl.ds((r0 + 2) * LANES, LANES)]
        lo = rec0[0] | rec1[0] | rec2[0]
        hi = rec0[1] | rec1[1] | rec2[1]
        j1 = rec0[3] * LVL + rec1[2]
        j2 = rec1[3] * LVL + rec2[2]
        lo, hi = _set_bit(lo, hi, j1)
        lo, hi = _set_bit(lo, hi, j2)
        s_last = rec2[3]
        lo_v = jnp.broadcast_to(lo, (LANES,))
        hi_v = jnp.broadcast_to(hi, (LANES,))
        cs = (_popcount32(lo_v) + _popcount32(hi_v)).astype(jnp.float32)

        def combine(pv, carry):
            rows = (NSROW + pv * LANES + iota) * LANES
            p_lo = plsc.load_gather(recs_v, [rows])
            p_hi = plsc.load_gather(recs_v, [rows + 1])
            p_first = plsc.load_gather(recs_v, [rows + 2])

            u_lo = p_lo | lo_v
            u_hi = p_hi | hi_v
            pc = _popcount32(u_lo) + _popcount32(u_hi)
            j = s_last * LVL + p_first
            bit = jnp.where(
                j < 32,
                lax.shift_right_logical(u_lo, jnp.minimum(j, 31)) & 1,
                lax.shift_right_logical(u_hi, jnp.maximum(j - 32, 0)) & 1,
            )
            csp = (pc + 1 - bit).astype(jnp.float32)
            cp = (_popcount32(p_lo) + _popcount32(p_hi)).astype(jnp.float32)
            ncd = (csp - jnp.minimum(cs, cp)) / jnp.maximum(cs, cp)
            stage_f[...] = ncd
            pltpu.sync_copy(stage_f, out_hbm.at[b, pl.ds(pv * LANES, LANES)])
            return carry

        lax.fori_loop(0, 4, combine, None)

    return ccp


@functools.cache
def _get_sc():
    return _make_sc()


def kernel(x, curve, levels, pmap):
    x48 = x.reshape(NSROW, N)
    lev48 = jnp.tile(levels.astype(jnp.float32), (B, 1))
    pf = pmap.reshape(-1, pmap.shape[-1]).astype(jnp.int32)
    q48 = _tc_quant(x48, lev48)
    return _get_sc()(q48, curve.astype(jnp.int32), pf)
